# Initial kernel scaffold; baseline (speedup 1.0000x reference)
import jax
import jax.numpy as jnp
from jax.experimental import pallas as pl


def kernel(x, y, edge_index_no_diag, edge_attr_no_diag, ybus, edge_index, edge_attr):
    raise NotImplementedError("write your pallas kernel here")


# speedup vs baseline: 36.2640x; 36.2640x over previous
import functools

import jax
import jax.numpy as jnp
from jax import lax
from jax.experimental import pallas as pl
from jax.experimental.pallas import tpu as pltpu
from jax.experimental.pallas import tpu_sc as plsc

N = 50000
NBUS = 500
NPAD = 51200
NT = 16
CN = NPAD // NT
NV = CN // 16
EA = 800000
C = 2000
CHA = EA // NT // C
E2 = 850000
EBP = 896000
CHB = EBP // NT // C
C16 = C // 16
NITER = 11

_mesh = plsc.VectorSubcoreMesh(
    core_axis_name="c", subcore_axis_name="s", num_cores=2, num_subcores=NT)


@functools.partial(
    pl.kernel,
    out_type=(
        jax.ShapeDtypeStruct((NPAD,), jnp.float32),
        jax.ShapeDtypeStruct((NITER * 16,), jnp.float32),
    ),
    mesh=_mesh,
    scratch_types=[
        pltpu.VMEM_SHARED((NPAD,), jnp.float32),
        pltpu.VMEM_SHARED((NPAD,), jnp.float32),
        pltpu.VMEM_SHARED((NPAD,), jnp.float32),
        pltpu.VMEM_SHARED((NPAD,), jnp.float32),
        pltpu.VMEM_SHARED((NT * 16,), jnp.float32),
        pltpu.VMEM((CN,), jnp.float32),
        pltpu.VMEM((CN,), jnp.float32),
        pltpu.VMEM((CN,), jnp.float32),
        pltpu.VMEM((CN,), jnp.int32),
        pltpu.VMEM((CN,), jnp.float32),
        pltpu.VMEM((CN,), jnp.float32),
        pltpu.VMEM((CN,), jnp.float32),
        pltpu.VMEM((CN,), jnp.float32),
        pltpu.VMEM((CN,), jnp.float32),
        pltpu.VMEM((C,), jnp.int32),
        pltpu.VMEM((C,), jnp.int32),
        pltpu.VMEM((C,), jnp.float32),
        pltpu.VMEM((C,), jnp.float32),
        pltpu.VMEM((C,), jnp.float32),
        pltpu.VMEM((NT * 16,), jnp.float32),
        pltpu.SemaphoreType.DMA,
    ],
)
def _gpg_sc(x0_h, x1_h, didx_h, ridx_h, yb_h,
            srcA_h, dstA_h, wA_h, srcB_h, dstB_h, wB_h,
            out_h, err_h,
            theta_sh, g_sh, aggr_sh, aggr2_sh, errp_sh,
            inp_v, den_v, dsafe_v, ridx_v, zero_v, acc_v, g_v, gref_v, th_v,
            srci_v, dsti_v, w_v, tv_v, msg_v, errall_v, sem):
    cid = lax.axis_index("c")
    sid = lax.axis_index("s")
    on0 = cid == 0
    base_n = sid * CN

    @pl.when(on0)
    def _phase0():
        pltpu.sync_copy(x0_h.at[pl.ds(base_n, CN)], inp_v)
        pltpu.sync_copy(x1_h.at[pl.ds(base_n, CN)], th_v)
        pltpu.sync_copy(didx_h.at[pl.ds(base_n, CN)], ridx_v)
        pltpu.async_copy(yb_h.at[ridx_v], den_v, sem).wait()
        pltpu.sync_copy(ridx_h.at[pl.ds(base_n, CN)], ridx_v)

        def f0(i, carry):
            s = pl.ds(i * 16, 16)
            inp_v[s] = inp_v[s] - th_v[s]
            d = den_v[s]
            dsafe_v[s] = jnp.where(d != 0.0, d, 1.0)
            zero_v[s] = jnp.zeros((16,), jnp.float32)
            return carry
        lax.fori_loop(0, NV, f0, 0)

    plsc.subcore_barrier()

    def mp_pass(src_h, dst_h, w_h, agg_sh, nchunks):
        ebase = sid * (nchunks * C)

        def chunk(cc, carry):
            b = ebase + cc * C
            pltpu.sync_copy(src_h.at[pl.ds(b, C)], srci_v)
            pltpu.sync_copy(dst_h.at[pl.ds(b, C)], dsti_v)
            pltpu.sync_copy(w_h.at[pl.ds(b, C)], w_v)
            pltpu.async_copy(theta_sh.at[srci_v], tv_v, sem).wait()

            def mul(i, c2):
                s = pl.ds(i * 16, 16)
                msg_v[s] = tv_v[s] * w_v[s]
                return c2
            lax.fori_loop(0, C16, mul, 0)
            pltpu.sync_copy(msg_v, agg_sh.at[dsti_v], add=True)
            return carry
        lax.fori_loop(0, nchunks, chunk, 0)

    def pass_body(k, carry):
        @pl.when(on0)
        def _zero():
            pltpu.sync_copy(zero_v, aggr_sh.at[pl.ds(base_n, CN)])
            pltpu.sync_copy(zero_v, aggr2_sh.at[pl.ds(base_n, CN)])
        plsc.subcore_barrier()

        @pl.when(on0 & (k > 0))
        def _mpa():
            mp_pass(srcA_h, dstA_h, wA_h, aggr_sh, CHA)
        plsc.subcore_barrier()

        @pl.when(on0)
        def _fin_g():
            pltpu.sync_copy(aggr_sh.at[pl.ds(base_n, CN)], acc_v)

            def fg(i, c2):
                s = pl.ds(i * 16, 16)
                d = den_v[s]
                g_v[s] = jnp.where(
                    d != 0.0, (inp_v[s] - acc_v[s]) / dsafe_v[s], 0.0)
                return c2
            lax.fori_loop(0, NV, fg, 0)
            pltpu.sync_copy(g_v, g_sh.at[pl.ds(base_n, CN)])
        plsc.subcore_barrier()

        @pl.when(on0)
        def _fin_theta():
            pltpu.async_copy(g_sh.at[ridx_v], gref_v, sem).wait()

            def ft(i, c2):
                s = pl.ds(i * 16, 16)
                d = den_v[s]
                th_v[s] = jnp.where(d == 0.0, 0.0, g_v[s] - gref_v[s])
                return c2
            lax.fori_loop(0, NV, ft, 0)
            pltpu.sync_copy(th_v, theta_sh.at[pl.ds(base_n, CN)])
        plsc.subcore_barrier()

        @pl.when(on0)
        def _mpb():
            mp_pass(srcB_h, dstB_h, wB_h, aggr2_sh, CHB)
        plsc.subcore_barrier()

        @pl.when(on0)
        def _err():
            pltpu.sync_copy(aggr2_sh.at[pl.ds(base_n, CN)], acc_v)
            lanes = lax.iota(jnp.int32, 16)

            def fe(i, acc):
                s = pl.ds(i * 16, 16)
                nid = (base_n + i * 16) + lanes
                v = jnp.abs(inp_v[s] - acc_v[s])
                return acc + jnp.where(nid < N, v, 0.0)
            tot = lax.fori_loop(0, NV, fe, jnp.zeros((16,), jnp.float32))
            gref_v[pl.ds(0, 16)] = tot
            pltpu.sync_copy(gref_v.at[pl.ds(0, 16)],
                            errp_sh.at[pl.ds(sid * 16, 16)])
        plsc.subcore_barrier()

        @pl.when(on0 & (sid == 0))
        def _reduce():
            pltpu.sync_copy(errp_sh, errall_v)

            def fr(i, acc):
                return acc + errall_v[pl.ds(i * 16, 16)]
            tot = lax.fori_loop(0, NT, fr, jnp.zeros((16,), jnp.float32))
            gref_v[pl.ds(0, 16)] = tot
            pltpu.sync_copy(gref_v.at[pl.ds(0, 16)],
                            err_h.at[pl.ds(k * 16, 16)])
        return carry

    lax.fori_loop(0, NITER, pass_body, 0)
    plsc.subcore_barrier()

    @pl.when(on0)
    def _out():
        pltpu.sync_copy(th_v, out_h.at[pl.ds(base_n, CN)])


def kernel(x, y, edge_index_no_diag, edge_attr_no_diag, ybus, edge_index,
           edge_attr):
    del y
    f32, i32 = jnp.float32, jnp.int32
    x0 = jnp.zeros((NPAD,), f32).at[:N].set(x[:, 0])
    x1 = jnp.zeros((NPAD,), f32).at[:N].set(x[:, 1])
    n_ar = jnp.arange(N, dtype=i32)
    didx = jnp.zeros((NPAD,), i32).at[:N].set(
        (n_ar // NBUS) * (NBUS * NBUS) + (n_ar % NBUS) * (NBUS + 1))
    ridx = jnp.zeros((NPAD,), i32).at[:N].set((n_ar // NBUS) * NBUS)
    yb_flat = ybus.reshape(-1)
    srcA = edge_index_no_diag[0]
    dstA = edge_index_no_diag[1]
    wA = edge_attr_no_diag
    srcB = jnp.zeros((EBP,), i32).at[:E2].set(edge_index[0])
    dstB = jnp.full((EBP,), N, i32).at[:E2].set(edge_index[1])
    wB = jnp.zeros((EBP,), f32).at[:E2].set(edge_attr)

    out_full, err_raw = _gpg_sc(x0, x1, didx, ridx, yb_flat,
                                srcA, dstA, wA, srcB, dstB, wB)
    out = out_full[:N].reshape(N, 1)
    errors = err_raw.reshape(NITER, 16).sum(axis=1)
    return out, errors


# speedup vs baseline: 53.1079x; 1.4645x over previous
import functools

import jax
import jax.numpy as jnp
from jax import lax
from jax.experimental import pallas as pl
from jax.experimental.pallas import tpu as pltpu
from jax.experimental.pallas import tpu_sc as plsc

N = 50000
NBUS = 500
NPAD = 51200
NT = 16
CN = NPAD // NT
NV = CN // 16
EA = 800000
C = 2000
EAP = 832000
CHA = EAP // NT // C
E2 = 850000
EBP = 896000
CHB = EBP // NT // C
C16 = C // 16
NITER = 11

_mesh = plsc.VectorSubcoreMesh(
    core_axis_name="c", subcore_axis_name="s", num_cores=2, num_subcores=NT)


@functools.partial(
    pl.kernel,
    out_type=(
        jax.ShapeDtypeStruct((NPAD,), jnp.float32),
        jax.ShapeDtypeStruct((NITER * 16,), jnp.float32),
    ),
    mesh=_mesh,
    scratch_types=[
        pltpu.VMEM_SHARED((NPAD,), jnp.float32),
        pltpu.VMEM_SHARED((NPAD,), jnp.float32),
        pltpu.VMEM_SHARED((NPAD,), jnp.float32),
        pltpu.VMEM_SHARED((NPAD,), jnp.float32),
        pltpu.VMEM_SHARED((NT * 16,), jnp.float32),
        pltpu.VMEM((CN,), jnp.float32),
        pltpu.VMEM((CN,), jnp.float32),
        pltpu.VMEM((CN,), jnp.float32),
        pltpu.VMEM((CN,), jnp.int32),
        pltpu.VMEM((CN,), jnp.float32),
        pltpu.VMEM((CN,), jnp.float32),
        pltpu.VMEM((CN,), jnp.float32),
        pltpu.VMEM((CN,), jnp.float32),
        pltpu.VMEM((CN,), jnp.float32),
        pltpu.VMEM((C,), jnp.int32),
        pltpu.VMEM((C,), jnp.int32),
        pltpu.VMEM((C,), jnp.int32),
        pltpu.VMEM((C,), jnp.int32),
        pltpu.VMEM((C,), jnp.float32),
        pltpu.VMEM((C,), jnp.float32),
        pltpu.VMEM((C,), jnp.float32),
        pltpu.VMEM((C,), jnp.float32),
        pltpu.VMEM((C,), jnp.float32),
        pltpu.VMEM((C,), jnp.float32),
        pltpu.VMEM((NT * 16,), jnp.float32),
        pltpu.SemaphoreType.DMA,
        pltpu.SemaphoreType.DMA,
        pltpu.SemaphoreType.DMA,
        pltpu.SemaphoreType.DMA,
        pltpu.SemaphoreType.DMA,
        pltpu.SemaphoreType.DMA,
    ],
)
def _gpg_sc(x0_h, x1_h, didx_h, ridx_h, yb_h,
            srcA_h, dstA_h, wA_h, srcB_h, dstB_h, wB_h,
            out_h, err_h,
            theta_sh, g_sh, aggr_sh, aggr2_sh, errp_sh,
            inp_v, den_v, dsafe_v, ridx_v, zero_v, acc_v, g_v, gref_v, th_v,
            srci0, srci1, dsti0, dsti1, w0, w1, tv0, tv1, msg0, msg1,
            errall_v, sem, lsem0, lsem1, gsem, ssem0, ssem1):
    cid = lax.axis_index("c")
    sid = lax.axis_index("s")
    on0 = cid == 0
    base_n = sid * CN

    @pl.when(on0)
    def _phase0():
        pltpu.sync_copy(x0_h.at[pl.ds(base_n, CN)], inp_v)
        pltpu.sync_copy(x1_h.at[pl.ds(base_n, CN)], th_v)
        pltpu.sync_copy(didx_h.at[pl.ds(base_n, CN)], ridx_v)
        pltpu.async_copy(yb_h.at[ridx_v], den_v, sem).wait()
        pltpu.sync_copy(ridx_h.at[pl.ds(base_n, CN)], ridx_v)

        def f0(i, carry):
            s = pl.ds(i * 16, 16)
            inp_v[s] = inp_v[s] - th_v[s]
            d = den_v[s]
            dsafe_v[s] = jnp.where(d != 0.0, d, 1.0)
            zero_v[s] = jnp.zeros((16,), jnp.float32)
            return carry
        lax.fori_loop(0, NV, f0, 0)

    plsc.subcore_barrier()

    def mp_pass(src_h, dst_h, w_h, agg_sh, nchunks):
        ebase = sid * (nchunks * C)
        bufs = ((srci0, dsti0, w0, tv0, msg0, lsem0, ssem0),
                (srci1, dsti1, w1, tv1, msg1, lsem1, ssem1))

        def fire_loads(b, p):
            si, di, wv, _, _, ls, _ = bufs[p]
            pltpu.async_copy(src_h.at[pl.ds(b, C)], si, ls)
            pltpu.async_copy(dst_h.at[pl.ds(b, C)], di, ls)
            pltpu.async_copy(w_h.at[pl.ds(b, C)], wv, ls)

        def drain_loads(p):
            si, di, wv, _, _, ls, _ = bufs[p]
            pltpu.make_async_copy(src_h.at[pl.ds(0, C)], si, ls).wait()
            pltpu.make_async_copy(dst_h.at[pl.ds(0, C)], di, ls).wait()
            pltpu.make_async_copy(w_h.at[pl.ds(0, C)], wv, ls).wait()

        def drain_scatter(p):
            _, _, _, _, mg, _, ss = bufs[p]
            pltpu.make_async_copy(w_h.at[pl.ds(0, C)], mg, ss).wait()

        def do_chunk(c2, p):
            si, di, wv, tv, mg, _, ss = bufs[p]
            drain_loads(p)
            pltpu.async_copy(theta_sh.at[si], tv, gsem).wait()

            @pl.when(c2 > 0)
            def _():
                drain_scatter(p)

            def mul(i, c3):
                s = pl.ds(i * 16, 16)
                mg[s] = tv[s] * wv[s]
                return c3
            lax.fori_loop(0, C16, mul, 0)
            pltpu.async_copy(mg, agg_sh.at[di], ss, add=True)

        fire_loads(ebase, 0)
        npairs = nchunks // 2

        def pair(c2, carry):
            b = ebase + (2 * c2) * C
            fire_loads(b + C, 1)
            do_chunk(c2, 0)

            @pl.when(c2 + 1 < npairs)
            def _():
                fire_loads(b + 2 * C, 0)
            do_chunk(c2, 1)
            return carry
        lax.fori_loop(0, npairs, pair, 0)
        drain_scatter(0)
        drain_scatter(1)

    def pass_body(k, carry):
        @pl.when(on0)
        def _zero():
            pltpu.sync_copy(zero_v, aggr_sh.at[pl.ds(base_n, CN)])
            pltpu.sync_copy(zero_v, aggr2_sh.at[pl.ds(base_n, CN)])
        plsc.subcore_barrier()

        @pl.when(on0 & (k > 0))
        def _mpa():
            mp_pass(srcA_h, dstA_h, wA_h, aggr_sh, CHA)
        plsc.subcore_barrier()

        @pl.when(on0)
        def _fin_g():
            pltpu.sync_copy(aggr_sh.at[pl.ds(base_n, CN)], acc_v)

            def fg(i, c2):
                s = pl.ds(i * 16, 16)
                d = den_v[s]
                g_v[s] = jnp.where(
                    d != 0.0, (inp_v[s] - acc_v[s]) / dsafe_v[s], 0.0)
                return c2
            lax.fori_loop(0, NV, fg, 0)
            pltpu.sync_copy(g_v, g_sh.at[pl.ds(base_n, CN)])
        plsc.subcore_barrier()

        @pl.when(on0)
        def _fin_theta():
            pltpu.async_copy(g_sh.at[ridx_v], gref_v, sem).wait()

            def ft(i, c2):
                s = pl.ds(i * 16, 16)
                d = den_v[s]
                th_v[s] = jnp.where(d == 0.0, 0.0, g_v[s] - gref_v[s])
                return c2
            lax.fori_loop(0, NV, ft, 0)
            pltpu.sync_copy(th_v, theta_sh.at[pl.ds(base_n, CN)])
        plsc.subcore_barrier()

        @pl.when(on0)
        def _mpb():
            mp_pass(srcB_h, dstB_h, wB_h, aggr2_sh, CHB)
        plsc.subcore_barrier()

        @pl.when(on0)
        def _err():
            pltpu.sync_copy(aggr2_sh.at[pl.ds(base_n, CN)], acc_v)
            lanes = lax.iota(jnp.int32, 16)

            def fe(i, acc):
                s = pl.ds(i * 16, 16)
                nid = (base_n + i * 16) + lanes
                v = jnp.abs(inp_v[s] - acc_v[s])
                return acc + jnp.where(nid < N, v, 0.0)
            tot = lax.fori_loop(0, NV, fe, jnp.zeros((16,), jnp.float32))
            gref_v[pl.ds(0, 16)] = tot
            pltpu.sync_copy(gref_v.at[pl.ds(0, 16)],
                            errp_sh.at[pl.ds(sid * 16, 16)])
        plsc.subcore_barrier()

        @pl.when(on0 & (sid == 0))
        def _reduce():
            pltpu.sync_copy(errp_sh, errall_v)

            def fr(i, acc):
                return acc + errall_v[pl.ds(i * 16, 16)]
            tot = lax.fori_loop(0, NT, fr, jnp.zeros((16,), jnp.float32))
            gref_v[pl.ds(0, 16)] = tot
            pltpu.sync_copy(gref_v.at[pl.ds(0, 16)],
                            err_h.at[pl.ds(k * 16, 16)])
        return carry

    lax.fori_loop(0, NITER, pass_body, 0)
    plsc.subcore_barrier()

    @pl.when(on0)
    def _out():
        pltpu.sync_copy(th_v, out_h.at[pl.ds(base_n, CN)])


def kernel(x, y, edge_index_no_diag, edge_attr_no_diag, ybus, edge_index,
           edge_attr):
    del y
    f32, i32 = jnp.float32, jnp.int32
    x0 = jnp.zeros((NPAD,), f32).at[:N].set(x[:, 0])
    x1 = jnp.zeros((NPAD,), f32).at[:N].set(x[:, 1])
    n_ar = jnp.arange(N, dtype=i32)
    didx = jnp.zeros((NPAD,), i32).at[:N].set(
        (n_ar // NBUS) * (NBUS * NBUS) + (n_ar % NBUS) * (NBUS + 1))
    ridx = jnp.zeros((NPAD,), i32).at[:N].set((n_ar // NBUS) * NBUS)
    yb_flat = ybus.reshape(-1)
    srcA = jnp.zeros((EAP,), i32).at[:EA].set(edge_index_no_diag[0])
    dstA = jnp.full((EAP,), N, i32).at[:EA].set(edge_index_no_diag[1])
    wA = jnp.zeros((EAP,), f32).at[:EA].set(edge_attr_no_diag)
    srcB = jnp.zeros((EBP,), i32).at[:E2].set(edge_index[0])
    dstB = jnp.full((EBP,), N, i32).at[:E2].set(edge_index[1])
    wB = jnp.zeros((EBP,), f32).at[:E2].set(edge_attr)

    out_full, err_raw = _gpg_sc(x0, x1, didx, ridx, yb_flat,
                                srcA, dstA, wA, srcB, dstB, wB)
    out = out_full[:N].reshape(N, 1)
    errors = err_raw.reshape(NITER, 16).sum(axis=1)
    return out, errors


# speedup vs baseline: 85.9689x; 1.6188x over previous
import functools

import jax
import jax.numpy as jnp
from jax import lax
from jax.experimental import pallas as pl
from jax.experimental.pallas import tpu as pltpu
from jax.experimental.pallas import tpu_sc as plsc

N = 50000
NBUS = 500
NB = 100
NPAD = 51200
NT = 16
CN = NPAD // NT
NV = CN // 16
EA = 800000
C = 2000
EAP = 832000
CHA = EAP // NT // C
E2 = 850000
EBP = 896000
CHB = EBP // NT // C
C16 = C // 16
NITER = 11

_mesh = plsc.VectorSubcoreMesh(
    core_axis_name="c", subcore_axis_name="s", num_cores=2, num_subcores=NT)


def _diag_body(yb_ref, out_ref):
    x = yb_ref[0]
    ii = lax.broadcasted_iota(jnp.int32, (NBUS, NBUS), 0)
    jj = lax.broadcasted_iota(jnp.int32, (NBUS, NBUS), 1)
    out_ref[0, 0, :] = jnp.sum(jnp.where(ii == jj, x, 0.0), axis=0)


def _extract_diag(ybus):
    return pl.pallas_call(
        _diag_body,
        grid=(NB,),
        in_specs=[pl.BlockSpec((1, NBUS, NBUS), lambda b: (b, 0, 0))],
        out_specs=pl.BlockSpec((1, 1, NBUS), lambda b: (b, 0, 0)),
        out_shape=jax.ShapeDtypeStruct((NB, 1, NBUS), jnp.float32),
    )(ybus)


@functools.partial(
    pl.kernel,
    out_type=(
        jax.ShapeDtypeStruct((NPAD,), jnp.float32),
        jax.ShapeDtypeStruct((NITER * 16,), jnp.float32),
    ),
    mesh=_mesh,
    scratch_types=[
        pltpu.VMEM_SHARED((NPAD,), jnp.float32),
        pltpu.VMEM_SHARED((NPAD,), jnp.float32),
        pltpu.VMEM_SHARED((NPAD,), jnp.float32),
        pltpu.VMEM_SHARED((NPAD,), jnp.float32),
        pltpu.VMEM_SHARED((NT * 16,), jnp.float32),
        pltpu.VMEM((CN,), jnp.float32),
        pltpu.VMEM((CN,), jnp.float32),
        pltpu.VMEM((CN,), jnp.float32),
        pltpu.VMEM((CN,), jnp.int32),
        pltpu.VMEM((CN,), jnp.float32),
        pltpu.VMEM((CN,), jnp.float32),
        pltpu.VMEM((CN,), jnp.float32),
        pltpu.VMEM((CN,), jnp.float32),
        pltpu.VMEM((CN,), jnp.float32),
        pltpu.VMEM((C,), jnp.int32),
        pltpu.VMEM((C,), jnp.int32),
        pltpu.VMEM((C,), jnp.int32),
        pltpu.VMEM((C,), jnp.int32),
        pltpu.VMEM((C,), jnp.float32),
        pltpu.VMEM((C,), jnp.float32),
        pltpu.VMEM((C,), jnp.float32),
        pltpu.VMEM((C,), jnp.float32),
        pltpu.VMEM((C,), jnp.float32),
        pltpu.VMEM((C,), jnp.float32),
        pltpu.VMEM((NT * 16,), jnp.float32),
        pltpu.SemaphoreType.DMA,
        pltpu.SemaphoreType.DMA,
        pltpu.SemaphoreType.DMA,
        pltpu.SemaphoreType.DMA,
        pltpu.SemaphoreType.DMA,
        pltpu.SemaphoreType.DMA,
    ],
)
def _gpg_sc(x0_h, x1_h, den_h, ridx_h,
            srcA_h, dstA_h, wA_h, srcB_h, dstB_h, wB_h,
            out_h, err_h,
            theta_sh, g_sh, aggr_sh, aggr2_sh, errp_sh,
            inp_v, den_v, dsafe_v, ridx_v, zero_v, acc_v, g_v, gref_v, th_v,
            srci0, srci1, dsti0, dsti1, w0, w1, tv0, tv1, msg0, msg1,
            errall_v, sem, lsem0, lsem1, gsem, ssem0, ssem1):
    cid = lax.axis_index("c")
    sid = lax.axis_index("s")
    on0 = cid == 0
    base_n = sid * CN

    @pl.when(on0)
    def _phase0():
        pltpu.sync_copy(x0_h.at[pl.ds(base_n, CN)], inp_v)
        pltpu.sync_copy(x1_h.at[pl.ds(base_n, CN)], th_v)
        pltpu.sync_copy(den_h.at[pl.ds(base_n, CN)], den_v)
        pltpu.sync_copy(ridx_h.at[pl.ds(base_n, CN)], ridx_v)

        def f0(i, carry):
            s = pl.ds(i * 16, 16)
            inp_v[s] = inp_v[s] - th_v[s]
            d = den_v[s]
            dsafe_v[s] = jnp.where(d != 0.0, d, 1.0)
            zero_v[s] = jnp.zeros((16,), jnp.float32)
            return carry
        lax.fori_loop(0, NV, f0, 0)

    plsc.subcore_barrier()

    def mp_pass(src_h, dst_h, w_h, agg_sh, nchunks):
        ebase = sid * (nchunks * C)
        bufs = ((srci0, dsti0, w0, tv0, msg0, lsem0, ssem0),
                (srci1, dsti1, w1, tv1, msg1, lsem1, ssem1))

        def fire_loads(b, p):
            si, di, wv, _, _, ls, _ = bufs[p]
            pltpu.async_copy(src_h.at[pl.ds(b, C)], si, ls)
            pltpu.async_copy(dst_h.at[pl.ds(b, C)], di, ls)
            pltpu.async_copy(w_h.at[pl.ds(b, C)], wv, ls)

        def drain_loads(p):
            si, di, wv, _, _, ls, _ = bufs[p]
            pltpu.make_async_copy(src_h.at[pl.ds(0, C)], si, ls).wait()
            pltpu.make_async_copy(dst_h.at[pl.ds(0, C)], di, ls).wait()
            pltpu.make_async_copy(w_h.at[pl.ds(0, C)], wv, ls).wait()

        def drain_scatter(p):
            _, _, _, _, mg, _, ss = bufs[p]
            pltpu.make_async_copy(w_h.at[pl.ds(0, C)], mg, ss).wait()

        def do_chunk(c2, p):
            si, di, wv, tv, mg, _, ss = bufs[p]
            drain_loads(p)
            pltpu.async_copy(theta_sh.at[si], tv, gsem).wait()

            @pl.when(c2 > 0)
            def _():
                drain_scatter(p)

            def mul(i, c3):
                s = pl.ds(i * 16, 16)
                mg[s] = tv[s] * wv[s]
                return c3
            lax.fori_loop(0, C16, mul, 0)
            pltpu.async_copy(mg, agg_sh.at[di], ss, add=True)

        fire_loads(ebase, 0)
        npairs = nchunks // 2

        def pair(c2, carry):
            b = ebase + (2 * c2) * C
            fire_loads(b + C, 1)
            do_chunk(c2, 0)

            @pl.when(c2 + 1 < npairs)
            def _():
                fire_loads(b + 2 * C, 0)
            do_chunk(c2, 1)
            return carry
        lax.fori_loop(0, npairs, pair, 0)
        drain_scatter(0)
        drain_scatter(1)

    def pass_body(k, carry):
        @pl.when(on0)
        def _zero():
            pltpu.sync_copy(zero_v, aggr_sh.at[pl.ds(base_n, CN)])
            pltpu.sync_copy(zero_v, aggr2_sh.at[pl.ds(base_n, CN)])
        plsc.subcore_barrier()

        @pl.when(on0 & (k > 0))
        def _mpa():
            mp_pass(srcA_h, dstA_h, wA_h, aggr_sh, CHA)
        plsc.subcore_barrier()

        @pl.when(on0)
        def _fin_g():
            pltpu.sync_copy(aggr_sh.at[pl.ds(base_n, CN)], acc_v)

            def fg(i, c2):
                s = pl.ds(i * 16, 16)
                d = den_v[s]
                g_v[s] = jnp.where(
                    d != 0.0, (inp_v[s] - acc_v[s]) / dsafe_v[s], 0.0)
                return c2
            lax.fori_loop(0, NV, fg, 0)
            pltpu.sync_copy(g_v, g_sh.at[pl.ds(base_n, CN)])
        plsc.subcore_barrier()

        @pl.when(on0)
        def _fin_theta():
            pltpu.async_copy(g_sh.at[ridx_v], gref_v, sem).wait()

            def ft(i, c2):
                s = pl.ds(i * 16, 16)
                d = den_v[s]
                th_v[s] = jnp.where(d == 0.0, 0.0, g_v[s] - gref_v[s])
                return c2
            lax.fori_loop(0, NV, ft, 0)
            pltpu.sync_copy(th_v, theta_sh.at[pl.ds(base_n, CN)])
        plsc.subcore_barrier()

        @pl.when(on0)
        def _mpb():
            mp_pass(srcB_h, dstB_h, wB_h, aggr2_sh, CHB)
        plsc.subcore_barrier()

        @pl.when(on0)
        def _err():
            pltpu.sync_copy(aggr2_sh.at[pl.ds(base_n, CN)], acc_v)
            lanes = lax.iota(jnp.int32, 16)

            def fe(i, acc):
                s = pl.ds(i * 16, 16)
                nid = (base_n + i * 16) + lanes
                v = jnp.abs(inp_v[s] - acc_v[s])
                return acc + jnp.where(nid < N, v, 0.0)
            tot = lax.fori_loop(0, NV, fe, jnp.zeros((16,), jnp.float32))
            gref_v[pl.ds(0, 16)] = tot
            pltpu.sync_copy(gref_v.at[pl.ds(0, 16)],
                            errp_sh.at[pl.ds(sid * 16, 16)])
        plsc.subcore_barrier()

        @pl.when(on0 & (sid == 0))
        def _reduce():
            pltpu.sync_copy(errp_sh, errall_v)

            def fr(i, acc):
                return acc + errall_v[pl.ds(i * 16, 16)]
            tot = lax.fori_loop(0, NT, fr, jnp.zeros((16,), jnp.float32))
            gref_v[pl.ds(0, 16)] = tot
            pltpu.sync_copy(gref_v.at[pl.ds(0, 16)],
                            err_h.at[pl.ds(k * 16, 16)])
        return carry

    lax.fori_loop(0, NITER, pass_body, 0)
    plsc.subcore_barrier()

    @pl.when(on0)
    def _out():
        pltpu.sync_copy(th_v, out_h.at[pl.ds(base_n, CN)])


def kernel(x, y, edge_index_no_diag, edge_attr_no_diag, ybus, edge_index,
           edge_attr):
    del y
    f32, i32 = jnp.float32, jnp.int32
    x0 = jnp.zeros((NPAD,), f32).at[:N].set(x[:, 0])
    x1 = jnp.zeros((NPAD,), f32).at[:N].set(x[:, 1])
    n_ar = jnp.arange(N, dtype=i32)
    ridx = jnp.zeros((NPAD,), i32).at[:N].set((n_ar // NBUS) * NBUS)
    den = jnp.zeros((NPAD,), f32).at[:N].set(_extract_diag(ybus).reshape(-1))
    srcA = jnp.zeros((EAP,), i32).at[:EA].set(edge_index_no_diag[0])
    dstA = jnp.full((EAP,), N, i32).at[:EA].set(edge_index_no_diag[1])
    wA = jnp.zeros((EAP,), f32).at[:EA].set(edge_attr_no_diag)
    srcB = jnp.zeros((EBP,), i32).at[:E2].set(edge_index[0])
    dstB = jnp.full((EBP,), N, i32).at[:E2].set(edge_index[1])
    wB = jnp.zeros((EBP,), f32).at[:E2].set(edge_attr)

    out_full, err_raw = _gpg_sc(x0, x1, den, ridx,
                                srcA, dstA, wA, srcB, dstB, wB)
    out = out_full[:N].reshape(N, 1)
    errors = err_raw.reshape(NITER, 16).sum(axis=1)
    return out, errors


# speedup vs baseline: 120.5701x; 1.4025x over previous
import functools

import jax
import jax.numpy as jnp
from jax import lax
from jax.experimental import pallas as pl
from jax.experimental.pallas import tpu as pltpu
from jax.experimental.pallas import tpu_sc as plsc

N = 50000
NBUS = 500
NB = 100
NPAD = 51200
NT = 16
CN = NPAD // NT
NV = CN // 16
EA = 800000
C = 2000
EAP = 832000
CHA = EAP // NT // C
E2 = 850000
EBP = 896000
CHB = EBP // NT // C
C16 = C // 16
NITER = 11
FBASE = 1100000001
ABASE = 1200000001
SPIN_LIMIT = 1024

_mesh = plsc.VectorSubcoreMesh(
    core_axis_name="c", subcore_axis_name="s", num_cores=2, num_subcores=NT)


def _diag_body(yb_ref, out_ref):
    x = yb_ref[0]
    ii = lax.broadcasted_iota(jnp.int32, (NBUS, NBUS), 0)
    jj = lax.broadcasted_iota(jnp.int32, (NBUS, NBUS), 1)
    out_ref[0, 0, :] = jnp.sum(jnp.where(ii == jj, x, 0.0), axis=0)


def _extract_diag(ybus):
    return pl.pallas_call(
        _diag_body,
        grid=(NB,),
        in_specs=[pl.BlockSpec((1, NBUS, NBUS), lambda b: (b, 0, 0))],
        out_specs=pl.BlockSpec((1, 1, NBUS), lambda b: (b, 0, 0)),
        out_shape=jax.ShapeDtypeStruct((NB, 1, NBUS), jnp.float32),
    )(ybus)


@functools.partial(
    pl.kernel,
    out_type=(
        jax.ShapeDtypeStruct((NPAD,), jnp.float32),
        jax.ShapeDtypeStruct((NPAD,), jnp.float32),
        jax.ShapeDtypeStruct((NITER * 16,), jnp.float32),
        jax.ShapeDtypeStruct((16,), jnp.int32),
        jax.ShapeDtypeStruct((16,), jnp.int32),
    ),
    mesh=_mesh,
    scratch_types=[
        pltpu.VMEM_SHARED((NPAD,), jnp.float32),
        pltpu.VMEM_SHARED((NPAD,), jnp.float32),
        pltpu.VMEM_SHARED((NPAD,), jnp.float32),
        pltpu.VMEM_SHARED((NT * 16,), jnp.float32),
        pltpu.VMEM((CN,), jnp.float32),
        pltpu.VMEM((CN,), jnp.float32),
        pltpu.VMEM((CN,), jnp.float32),
        pltpu.VMEM((CN,), jnp.int32),
        pltpu.VMEM((CN,), jnp.float32),
        pltpu.VMEM((CN,), jnp.float32),
        pltpu.VMEM((CN,), jnp.float32),
        pltpu.VMEM((CN,), jnp.float32),
        pltpu.VMEM((CN,), jnp.float32),
        pltpu.VMEM((C,), jnp.int32),
        pltpu.VMEM((C,), jnp.int32),
        pltpu.VMEM((C,), jnp.int32),
        pltpu.VMEM((C,), jnp.int32),
        pltpu.VMEM((C,), jnp.float32),
        pltpu.VMEM((C,), jnp.float32),
        pltpu.VMEM((C,), jnp.float32),
        pltpu.VMEM((C,), jnp.float32),
        pltpu.VMEM((C,), jnp.float32),
        pltpu.VMEM((C,), jnp.float32),
        pltpu.VMEM((NT * 16,), jnp.float32),
        pltpu.VMEM((16,), jnp.int32),
        pltpu.SemaphoreType.DMA,
        pltpu.SemaphoreType.DMA,
        pltpu.SemaphoreType.DMA,
        pltpu.SemaphoreType.DMA,
        pltpu.SemaphoreType.DMA,
        pltpu.SemaphoreType.DMA,
    ],
)
def _gpg_sc(x0_h, x1_h, den_h, ridx_h,
            srcA_h, dstA_h, wA_h, srcB_h, dstB_h, wB_h,
            th0_h, th1_h, err_h, flag_h, ack_h,
            theta_sh, g_sh, aggr_sh, errp_sh,
            inp_v, den_v, dsafe_v, ridx_v, zero_v, acc_v, g_v, gref_v, th_v,
            srci0, srci1, dsti0, dsti1, w0, w1, tv0, tv1, msg0, msg1,
            errall_v, fbuf_v, sem, lsem0, lsem1, gsem, ssem0, ssem1):
    cid = lax.axis_index("c")
    sid = lax.axis_index("s")
    base_n = sid * CN

    def spin(mb_h, t1, t2):
        def body(i, v):
            done = (v == t1) | (v == t2)

            @pl.when(jnp.logical_not(done))
            def _():
                pltpu.sync_copy(mb_h, fbuf_v)
            return fbuf_v[pl.ds(0, 16)][0]
        lax.fori_loop(0, SPIN_LIMIT, body, jnp.int32(-1))

    def post(mb_h, val):
        fbuf_v[pl.ds(0, 16)] = jnp.full((16,), val, jnp.int32)
        pltpu.sync_copy(fbuf_v, mb_h)

    def mp_pass(src_h, dst_h, w_h, nchunks):
        ebase = sid * (nchunks * C)
        bufs = ((srci0, dsti0, w0, tv0, msg0, lsem0, ssem0),
                (srci1, dsti1, w1, tv1, msg1, lsem1, ssem1))

        def fire_loads(b, p):
            si, di, wv, _, _, ls, _ = bufs[p]
            pltpu.async_copy(src_h.at[pl.ds(b, C)], si, ls)
            pltpu.async_copy(dst_h.at[pl.ds(b, C)], di, ls)
            pltpu.async_copy(w_h.at[pl.ds(b, C)], wv, ls)

        def drain_loads(p):
            si, di, wv, _, _, ls, _ = bufs[p]
            pltpu.make_async_copy(src_h.at[pl.ds(0, C)], si, ls).wait()
            pltpu.make_async_copy(dst_h.at[pl.ds(0, C)], di, ls).wait()
            pltpu.make_async_copy(w_h.at[pl.ds(0, C)], wv, ls).wait()

        def drain_scatter(p):
            _, _, _, _, mg, _, ss = bufs[p]
            pltpu.make_async_copy(w_h.at[pl.ds(0, C)], mg, ss).wait()

        def do_chunk(c2, p):
            si, di, wv, tv, mg, _, ss = bufs[p]
            drain_loads(p)
            pltpu.async_copy(theta_sh.at[si], tv, gsem).wait()

            @pl.when(c2 > 0)
            def _():
                drain_scatter(p)

            def mul(i, c3):
                s = pl.ds(i * 16, 16)
                mg[s] = tv[s] * wv[s]
                return c3
            lax.fori_loop(0, C16, mul, 0)
            pltpu.async_copy(mg, aggr_sh.at[di], ss, add=True)

        fire_loads(ebase, 0)
        npairs = nchunks // 2

        def pair(c2, carry):
            b = ebase + (2 * c2) * C
            fire_loads(b + C, 1)
            do_chunk(c2, 0)

            @pl.when(c2 + 1 < npairs)
            def _():
                fire_loads(b + 2 * C, 0)
            do_chunk(c2, 1)
            return carry
        lax.fori_loop(0, npairs, pair, 0)
        drain_scatter(0)
        drain_scatter(1)

    def zero_aggr():
        pltpu.sync_copy(zero_v, aggr_sh.at[pl.ds(base_n, CN)])

    def load_inp_zero():
        pltpu.sync_copy(x0_h.at[pl.ds(base_n, CN)], inp_v)
        pltpu.sync_copy(x1_h.at[pl.ds(base_n, CN)], th_v)

        def f0(i, carry):
            s = pl.ds(i * 16, 16)
            inp_v[s] = inp_v[s] - th_v[s]
            zero_v[s] = jnp.zeros((16,), jnp.float32)
            return carry
        lax.fori_loop(0, NV, f0, 0)

    @pl.when(cid == 0)
    def _core0():
        load_inp_zero()
        pltpu.sync_copy(den_h.at[pl.ds(base_n, CN)], den_v)
        pltpu.sync_copy(ridx_h.at[pl.ds(base_n, CN)], ridx_v)

        def fd(i, carry):
            s = pl.ds(i * 16, 16)
            d = den_v[s]
            dsafe_v[s] = jnp.where(d != 0.0, d, 1.0)
            return carry
        lax.fori_loop(0, NV, fd, 0)
        plsc.subcore_barrier()

        def pass0(k, carry):
            zero_aggr()
            plsc.subcore_barrier()

            @pl.when(k > 0)
            def _():
                mp_pass(srcA_h, dstA_h, wA_h, CHA)
            plsc.subcore_barrier()

            pltpu.sync_copy(aggr_sh.at[pl.ds(base_n, CN)], acc_v)

            def fg(i, c2):
                s = pl.ds(i * 16, 16)
                d = den_v[s]
                g_v[s] = jnp.where(
                    d != 0.0, (inp_v[s] - acc_v[s]) / dsafe_v[s], 0.0)
                return c2
            lax.fori_loop(0, NV, fg, 0)
            pltpu.sync_copy(g_v, g_sh.at[pl.ds(base_n, CN)])
            plsc.subcore_barrier()

            pltpu.async_copy(g_sh.at[ridx_v], gref_v, sem).wait()

            def ft(i, c2):
                s = pl.ds(i * 16, 16)
                d = den_v[s]
                th_v[s] = jnp.where(d == 0.0, 0.0, g_v[s] - gref_v[s])
                return c2
            lax.fori_loop(0, NV, ft, 0)
            pltpu.sync_copy(th_v, theta_sh.at[pl.ds(base_n, CN)])

            @pl.when((k >= 2) & (sid == 0))
            def _():
                spin(ack_h, ABASE + k - 1, ABASE + k)
            plsc.subcore_barrier()

            @pl.when(lax.rem(k + 1, 2) == 0)
            def _():
                pltpu.sync_copy(th_v, th0_h.at[pl.ds(base_n, CN)])

            @pl.when(lax.rem(k + 1, 2) == 1)
            def _():
                pltpu.sync_copy(th_v, th1_h.at[pl.ds(base_n, CN)])
            plsc.subcore_barrier()

            @pl.when(sid == 0)
            def _():
                post(flag_h, FBASE + k + 1)
            return carry
        lax.fori_loop(0, NITER, pass0, 0)

    @pl.when(cid == 1)
    def _core1():
        load_inp_zero()
        plsc.subcore_barrier()

        def pass1(k, carry):
            zero_aggr()
            plsc.subcore_barrier()

            @pl.when(sid == 0)
            def _():
                spin(flag_h, FBASE + k + 1, FBASE + k + 1)
            plsc.subcore_barrier()

            @pl.when(lax.rem(k + 1, 2) == 0)
            def _():
                pltpu.sync_copy(th0_h.at[pl.ds(base_n, CN)], th_v)

            @pl.when(lax.rem(k + 1, 2) == 1)
            def _():
                pltpu.sync_copy(th1_h.at[pl.ds(base_n, CN)], th_v)
            pltpu.sync_copy(th_v, theta_sh.at[pl.ds(base_n, CN)])
            plsc.subcore_barrier()

            @pl.when(sid == 0)
            def _():
                post(ack_h, ABASE + k + 1)
            mp_pass(srcB_h, dstB_h, wB_h, CHB)
            plsc.subcore_barrier()

            pltpu.sync_copy(aggr_sh.at[pl.ds(base_n, CN)], acc_v)
            lanes = lax.iota(jnp.int32, 16)

            def fe(i, acc):
                s = pl.ds(i * 16, 16)
                nid = (base_n + i * 16) + lanes
                v = jnp.abs(inp_v[s] - acc_v[s])
                return acc + jnp.where(nid < N, v, 0.0)
            tot = lax.fori_loop(0, NV, fe, jnp.zeros((16,), jnp.float32))
            gref_v[pl.ds(0, 16)] = tot
            pltpu.sync_copy(gref_v.at[pl.ds(0, 16)],
                            errp_sh.at[pl.ds(sid * 16, 16)])
            plsc.subcore_barrier()

            @pl.when(sid == 0)
            def _():
                pltpu.sync_copy(errp_sh, errall_v)

                def fr(i, acc):
                    return acc + errall_v[pl.ds(i * 16, 16)]
                t2 = lax.fori_loop(0, NT, fr, jnp.zeros((16,), jnp.float32))
                gref_v[pl.ds(0, 16)] = t2
                pltpu.sync_copy(gref_v.at[pl.ds(0, 16)],
                                err_h.at[pl.ds(k * 16, 16)])
            plsc.subcore_barrier()
            return carry
        lax.fori_loop(0, NITER, pass1, 0)


def kernel(x, y, edge_index_no_diag, edge_attr_no_diag, ybus, edge_index,
           edge_attr):
    del y
    f32, i32 = jnp.float32, jnp.int32
    x0 = jnp.zeros((NPAD,), f32).at[:N].set(x[:, 0])
    x1 = jnp.zeros((NPAD,), f32).at[:N].set(x[:, 1])
    n_ar = jnp.arange(N, dtype=i32)
    ridx = jnp.zeros((NPAD,), i32).at[:N].set((n_ar // NBUS) * NBUS)
    den = jnp.zeros((NPAD,), f32).at[:N].set(_extract_diag(ybus).reshape(-1))
    srcA = jnp.zeros((EAP,), i32).at[:EA].set(edge_index_no_diag[0])
    dstA = jnp.full((EAP,), N, i32).at[:EA].set(edge_index_no_diag[1])
    wA = jnp.zeros((EAP,), f32).at[:EA].set(edge_attr_no_diag)
    srcB = jnp.zeros((EBP,), i32).at[:E2].set(edge_index[0])
    dstB = jnp.full((EBP,), N, i32).at[:E2].set(edge_index[1])
    wB = jnp.zeros((EBP,), f32).at[:E2].set(edge_attr)

    _, out_full, err_raw, _, _ = _gpg_sc(x0, x1, den, ridx,
                                         srcA, dstA, wA, srcB, dstB, wB)
    out = out_full[:N].reshape(N, 1)
    errors = err_raw.reshape(NITER, 16).sum(axis=1)
    return out, errors
